# baseline (speedup 1.0000x reference)
import functools

import jax
import jax.numpy as jnp
from jax import lax
from jax.experimental import pallas as pl
from jax.experimental.pallas import tpu as pltpu
from jax.experimental.pallas import tpu_sc as plsc

NC = 2
NS = 16
NW = NC * NS
CHUNK = 128
NBUF = 4


def _ceil_to(a, m):
    return (a + m - 1) // m * m


def _mesh():
    return plsc.VectorSubcoreMesh(
        core_axis_name="c", subcore_axis_name="s",
        num_cores=NC, num_subcores=NS)


def _newton_rsqrt16(d):
    i = plsc.bitcast(d, jnp.int32)
    i = 0x5F3759DF - lax.shift_right_logical(i, 1)
    y = plsc.bitcast(i, jnp.float32)
    for _ in range(3):
        y = y * (1.5 - 0.5 * d * y * y)
    return y


def _iota16():
    return lax.iota(jnp.int32, 16)


def _splat(vec_ref, r):
    return plsc.load_gather(vec_ref, [jnp.full((16,), r, jnp.int32)])


def _zero_fill(buf, width):
    def fill(r, carry):
        for j in range(width // 16):
            buf[r, pl.ds(j * 16, 16)] = jnp.zeros((16,), jnp.float32)
        return carry
    lax.fori_loop(0, CHUNK, fill, 0)


def _compute_dinv(deg_v, dinv_v, rpt):
    def grp(g, carry):
        rows = g * 16 + _iota16()
        dvec = plsc.load_gather(deg_v, [rows, _iota16()])
        dinv_v[pl.ds(g * 16, 16)] = _newton_rsqrt16(dvec + 1.0)
        return carry
    lax.fori_loop(0, rpt // 16, grp, 0)


def _emit_agg(tab, src_v, dst_v, rows_v, acc_sh, gsems, ssems, epc):
    def gstart(c, b):
        pltpu.async_copy(tab.at[src_v.at[c]], rows_v.at[b], gsems[b])

    def gwait(c, b):
        pltpu.make_async_copy(tab.at[src_v.at[c]], rows_v.at[b],
                              gsems[b]).wait()

    def sstart(c, b):
        pltpu.async_copy(rows_v.at[b], acc_sh.at[dst_v.at[c]],
                         ssems[b], add=True)

    def swait(c, b):
        pltpu.make_async_copy(rows_v.at[b], acc_sh.at[dst_v.at[c]],
                              ssems[b]).wait()

    for b in range(NBUF):
        gstart(b, b)

    def round_body(i, carry):
        for b in range(NBUF):
            c = NBUF * i + b
            gwait(c, b)
            sstart(c, b)
        for b in range(NBUF):
            c = NBUF * i + b

            @pl.when(c + NBUF < epc)
            def _():
                swait(c, b)
                gstart(c + NBUF, b)
        return carry
    lax.fori_loop(0, epc // NBUF, round_body, 0)
    for b in range(NBUF):
        swait(epc - NBUF + b, b)


def _sc_a_kernel(npad, epc):
    rpt = npad // NS
    zch = rpt // CHUNK
    h = 64

    @functools.partial(
        pl.kernel,
        out_type=(
            jax.ShapeDtypeStruct((NC, npad, h), jnp.float32),
            jax.ShapeDtypeStruct((NC, npad, h), jnp.float32),
            jax.ShapeDtypeStruct((NC, npad, 16), jnp.float32),
        ),
        mesh=_mesh(),
        compiler_params=pltpu.CompilerParams(use_tc_tiling_on_sc=False, needs_layout_passes=False),
        scratch_types=[
            pltpu.VMEM((epc, CHUNK), jnp.int32),
            pltpu.VMEM((epc, CHUNK), jnp.int32),
            pltpu.VMEM((epc, CHUNK), jnp.int32),
            pltpu.VMEM((NBUF, CHUNK, h), jnp.float32),
            pltpu.VMEM((CHUNK, 16), jnp.float32),
            pltpu.VMEM((CHUNK, 16), jnp.float32),
            pltpu.VMEM((rpt, 16), jnp.float32),
            pltpu.VMEM((rpt,), jnp.float32),
            pltpu.VMEM_SHARED((npad, 16), jnp.float32),
            pltpu.VMEM_SHARED((npad, h), jnp.float32),
            pltpu.SemaphoreType.DMA,
            pltpu.SemaphoreType.DMA,
        ] + [pltpu.SemaphoreType.DMA] * (2 * NBUF),
    )
    def sca(xw_hbm, src_hbm, dst_hbm, agg_hbm, y1_hbm, deg_hbm,
            src_v, dst_v, dstp_v, rows_v, ones_v, zero16_v, deg_v, dinv_v,
            acc16_sh, acc64_sh, lsem, hsem, *sems):
        cid = lax.axis_index("c")
        sid = lax.axis_index("s")
        wid = sid * NC + cid
        pwid = sid * NC + (1 - cid)
        gsems = sems[:NBUF]
        ssems = sems[NBUF:]
        base = sid * rpt

        pltpu.async_copy(src_hbm.at[wid], src_v, lsem)
        pltpu.async_copy(dst_hbm.at[wid], dst_v, lsem)
        pltpu.async_copy(dst_hbm.at[pwid], dstp_v, lsem)

        def fill(r, carry):
            ones_v[r, :] = jnp.full((16,), 1.0, jnp.float32)
            zero16_v[r, :] = jnp.zeros((16,), jnp.float32)
            return carry
        lax.fori_loop(0, CHUNK, fill, 0)
        _zero_fill(rows_v.at[0], h)

        for z in range(zch):
            pltpu.sync_copy(zero16_v,
                            acc16_sh.at[pl.ds(base + z * CHUNK, CHUNK)])
            pltpu.sync_copy(rows_v.at[0],
                            acc64_sh.at[pl.ds(base + z * CHUNK, CHUNK)])
        pltpu.make_async_copy(src_hbm.at[wid], src_v, lsem).wait()
        pltpu.make_async_copy(dst_hbm.at[wid], dst_v, lsem).wait()
        pltpu.make_async_copy(dst_hbm.at[pwid], dstp_v, lsem).wait()
        plsc.subcore_barrier()

        for slab in (dst_v, dstp_v):
            def group(i, carry):
                for j in range(4):
                    pltpu.async_copy(ones_v, acc16_sh.at[slab.at[i * 4 + j]],
                                     hsem, add=True)
                for j in range(4):
                    pltpu.make_async_copy(
                        ones_v, acc16_sh.at[slab.at[i * 4 + j]], hsem).wait()
                return carry
            lax.fori_loop(0, epc // 4, group, 0)
        plsc.subcore_barrier()

        pltpu.sync_copy(acc16_sh.at[pl.ds(base, rpt)], deg_v)
        pltpu.sync_copy(deg_v, deg_hbm.at[cid, pl.ds(base, rpt)])
        _compute_dinv(deg_v, dinv_v, rpt)

        for z in range(zch):
            blk = rows_v.at[1]
            pltpu.sync_copy(xw_hbm.at[pl.ds(base + z * CHUNK, CHUNK)], blk)

            def scale(r, carry):
                sv = _splat(dinv_v, z * CHUNK + r)
                for q in range(h // 16):
                    blk[r, pl.ds(q * 16, 16)] = blk[r, pl.ds(q * 16, 16)] * sv
                return carry
            lax.fori_loop(0, CHUNK, scale, 0)
            pltpu.sync_copy(blk, y1_hbm.at[cid, pl.ds(base + z * CHUNK, CHUNK)])
        plsc.subcore_barrier()

        _emit_agg(y1_hbm.at[cid], src_v, dst_v, rows_v, acc64_sh,
                  gsems, ssems, epc)
        plsc.subcore_barrier()
        pltpu.sync_copy(acc64_sh.at[pl.ds(base, rpt)],
                        agg_hbm.at[cid, pl.ds(base, rpt)])

    return sca


def _sc_b_kernel(npad, epc):
    rpt = npad // NS
    zch = rpt // CHUNK
    h = 64

    @functools.partial(
        pl.kernel,
        out_type=(
            jax.ShapeDtypeStruct((NC, npad, h), jnp.float32),
            jax.ShapeDtypeStruct((NC, npad, h), jnp.float32),
        ),
        mesh=_mesh(),
        compiler_params=pltpu.CompilerParams(use_tc_tiling_on_sc=False, needs_layout_passes=False),
        scratch_types=[
            pltpu.VMEM((epc, CHUNK), jnp.int32),
            pltpu.VMEM((epc, CHUNK), jnp.int32),
            pltpu.VMEM((NBUF, CHUNK, h), jnp.float32),
            pltpu.VMEM((CHUNK, h), jnp.float32),
            pltpu.VMEM((CHUNK, h), jnp.float32),
            pltpu.VMEM((1, h), jnp.float32),
            pltpu.VMEM((rpt, 16), jnp.float32),
            pltpu.VMEM((rpt,), jnp.float32),
            pltpu.VMEM_SHARED((npad, h), jnp.float32),
            pltpu.SemaphoreType.DMA,
        ] + [pltpu.SemaphoreType.DMA] * (2 * NBUF),
    )
    def scb(a0_hbm, a1_hbm, y1_hbm, deg_hbm, b1_hbm, src_hbm, dst_hbm,
            agg_hbm, y2_hbm,
            src_v, dst_v, rows_v, sa_v, sy_v, b1_v, deg_v, dinv_v,
            acc64_sh, lsem, *sems):
        cid = lax.axis_index("c")
        sid = lax.axis_index("s")
        wid = sid * NC + cid
        gsems = sems[:NBUF]
        ssems = sems[NBUF:]
        base = sid * rpt

        pltpu.async_copy(src_hbm.at[wid], src_v, lsem)
        pltpu.async_copy(dst_hbm.at[wid], dst_v, lsem)
        pltpu.sync_copy(b1_hbm, b1_v)
        _zero_fill(rows_v.at[0], h)
        for z in range(zch):
            pltpu.sync_copy(rows_v.at[0],
                            acc64_sh.at[pl.ds(base + z * CHUNK, CHUNK)])

        pltpu.sync_copy(deg_hbm.at[pl.ds(base, rpt)], deg_v)
        _compute_dinv(deg_v, dinv_v, rpt)

        bvec = [b1_v[0, pl.ds(q * 16, 16)] for q in range(h // 16)]
        for z in range(zch):
            blk = rows_v.at[1]
            rows = pl.ds(base + z * CHUNK, CHUNK)
            pltpu.sync_copy(a0_hbm.at[rows], blk)
            pltpu.sync_copy(a1_hbm.at[rows], sa_v)
            pltpu.sync_copy(y1_hbm.at[rows], sy_v)

            def act(r, carry):
                sv = _splat(dinv_v, z * CHUNK + r)
                for q in range(h // 16):
                    sl = pl.ds(q * 16, 16)
                    u = (blk[r, sl] + sa_v[r, sl] + sy_v[r, sl]) * sv + bvec[q]
                    blk[r, sl] = jnp.maximum(u, 0.0) * sv
                return carry
            lax.fori_loop(0, CHUNK, act, 0)
            pltpu.sync_copy(blk, y2_hbm.at[cid, rows])

        pltpu.make_async_copy(src_hbm.at[wid], src_v, lsem).wait()
        pltpu.make_async_copy(dst_hbm.at[wid], dst_v, lsem).wait()
        plsc.subcore_barrier()

        _emit_agg(y2_hbm.at[cid], src_v, dst_v, rows_v, acc64_sh,
                  gsems, ssems, epc)
        plsc.subcore_barrier()
        pltpu.sync_copy(acc64_sh.at[pl.ds(base, rpt)],
                        agg_hbm.at[cid, pl.ds(base, rpt)])

    return scb


def _tc1_body(x_ref, w_ref, o_ref):
    o_ref[...] = jnp.dot(x_ref[...], w_ref[...],
                         preferred_element_type=jnp.float32)


def _tc2_body(a0_ref, a1_ref, y2_ref, d_ref, wc_ref, bc_ref, o_ref):
    dinv = lax.rsqrt(d_ref[:, 0:1] + 1.0)
    z = dinv * (a0_ref[...] + a1_ref[...] + y2_ref[...])
    o_ref[...] = (jnp.dot(z, wc_ref[...], preferred_element_type=jnp.float32)
                  + bc_ref[...])


def kernel(x, edge_index, W1, b1, Wmu, bmu, Wls, bls):
    n, d_in = x.shape
    h_dim = W1.shape[1]
    out_dim = Wmu.shape[1]
    e = edge_index.shape[1]

    npad = _ceil_to(n + CHUNK, NS * CHUNK)
    epw = _ceil_to(-(-e // NW), 4 * CHUNK)
    epc = epw // CHUNK
    epad = epw * NW

    pad_idx = (n + (jnp.arange(epad - e, dtype=jnp.int32) % CHUNK))
    srcp = jnp.concatenate([edge_index[0], pad_idx]).reshape(NW, epc, CHUNK)
    dstp = jnp.concatenate([edge_index[1], pad_idx]).reshape(NW, epc, CHUNK)
    x_pad = jnp.zeros((npad, d_in), x.dtype).at[:n].set(x)

    br = 2048
    grid = (npad // br,)
    row_spec = lambda width: pl.BlockSpec((br, width), lambda i: (i, 0))
    full_spec = lambda shape: pl.BlockSpec(shape, lambda i: (0, 0))

    xw = pl.pallas_call(
        _tc1_body,
        grid=grid,
        in_specs=[row_spec(d_in), full_spec((d_in, h_dim))],
        out_specs=row_spec(h_dim),
        out_shape=jax.ShapeDtypeStruct((npad, h_dim), jnp.float32),
    )(x_pad, W1)

    a1p, y1c, degc = _sc_a_kernel(npad, epc)(xw, srcp, dstp)
    a2p, y2c = _sc_b_kernel(npad, epc)(
        a1p[0], a1p[1], y1c[0], degc[0], b1.reshape(1, h_dim), srcp, dstp)

    wc = jnp.concatenate([Wmu, Wls], axis=1)
    bc = jnp.concatenate([bmu, bls]).reshape(1, 2 * out_dim)
    out = pl.pallas_call(
        _tc2_body,
        grid=grid,
        in_specs=[row_spec(h_dim), row_spec(h_dim), row_spec(h_dim),
                  row_spec(16),
                  full_spec((h_dim, 2 * out_dim)), full_spec((1, 2 * out_dim))],
        out_specs=row_spec(2 * out_dim),
        out_shape=jax.ShapeDtypeStruct((npad, 2 * out_dim), jnp.float32),
    )(a2p[0], a2p[1], y2c[0], degc[0], wc, bc)

    return out[:n, :out_dim], out[:n, out_dim:]


# speedup vs baseline: 1.1937x; 1.1937x over previous
import functools

import jax
import jax.numpy as jnp
from jax import lax
from jax.experimental import pallas as pl
from jax.experimental.pallas import tpu as pltpu
from jax.experimental.pallas import tpu_sc as plsc

NC = 2
NS = 16
NW = NC * NS
CHUNK = 128
NBUF = 4

_SC_PARAMS = pltpu.CompilerParams(use_tc_tiling_on_sc=False,
                                  needs_layout_passes=False)


def _ceil_to(a, m):
    return (a + m - 1) // m * m


def _mesh():
    return plsc.VectorSubcoreMesh(
        core_axis_name="c", subcore_axis_name="s",
        num_cores=NC, num_subcores=NS)


def _newton_rsqrt16(d):
    i = plsc.bitcast(d, jnp.int32)
    i = 0x5F3759DF - lax.shift_right_logical(i, 1)
    y = plsc.bitcast(i, jnp.float32)
    for _ in range(3):
        y = y * (1.5 - 0.5 * d * y * y)
    return y


def _iota16():
    return lax.iota(jnp.int32, 16)


def _splat(vec_ref, r):
    return plsc.load_gather(vec_ref, [jnp.full((16,), r, jnp.int32)])


def _zero_fill(buf, width):
    def fill(r, carry):
        for j in range(width // 16):
            buf[r, pl.ds(j * 16, 16)] = jnp.zeros((16,), jnp.float32)
        return carry
    lax.fori_loop(0, CHUNK, fill, 0)


def _compute_dinv(deg_v, dinv_v, rpt):
    def grp(g, carry):
        rows = g * 16 + _iota16()
        dvec = plsc.load_gather(deg_v, [rows, _iota16()])
        dinv_v[pl.ds(g * 16, 16)] = _newton_rsqrt16(dvec + 1.0)
        return carry
    lax.fori_loop(0, rpt // 16, grp, 0)


def _emit_agg(tab, src_v, dst_v, rows_v, acc_sh, gsems, ssems, epc):
    def gstart(c, b):
        pltpu.async_copy(tab.at[src_v.at[c]], rows_v.at[b], gsems[b])

    def gwait(c, b):
        pltpu.make_async_copy(tab.at[src_v.at[c]], rows_v.at[b],
                              gsems[b]).wait()

    def sstart(c, b):
        pltpu.async_copy(rows_v.at[b], acc_sh.at[dst_v.at[c]],
                         ssems[b], add=True)

    def swait(c, b):
        pltpu.make_async_copy(rows_v.at[b], acc_sh.at[dst_v.at[c]],
                              ssems[b]).wait()

    for b in range(NBUF):
        gstart(b, b)

    def round_body(i, carry):
        for b in range(NBUF):
            c = NBUF * i + b
            gwait(c, b)
            sstart(c, b)
        for b in range(NBUF):
            c = NBUF * i + b

            @pl.when(c + NBUF < epc)
            def _():
                swait(c, b)
                gstart(c + NBUF, b)
        return carry
    lax.fori_loop(0, epc // NBUF, round_body, 0)
    for b in range(NBUF):
        swait(epc - NBUF + b, b)


def _sc_a_kernel(npad, epc, h):
    rpt = npad // NS
    zch = rpt // CHUNK

    @functools.partial(
        pl.kernel,
        out_type=(
            jax.ShapeDtypeStruct((NC, npad, h), jnp.float32),
            jax.ShapeDtypeStruct((NC, npad, h), jnp.float32),
            jax.ShapeDtypeStruct((NC, npad, 16), jnp.float32),
        ),
        mesh=_mesh(),
        compiler_params=_SC_PARAMS,
        scratch_types=[
            pltpu.VMEM((epc, CHUNK), jnp.int32),
            pltpu.VMEM((epc, CHUNK), jnp.int32),
            pltpu.VMEM((epc, CHUNK), jnp.int32),
            pltpu.VMEM((NBUF, CHUNK, h), jnp.float32),
            pltpu.VMEM((CHUNK, 16), jnp.float32),
            pltpu.VMEM((CHUNK, 16), jnp.float32),
            pltpu.VMEM((rpt, 16), jnp.float32),
            pltpu.VMEM((rpt,), jnp.float32),
            pltpu.VMEM_SHARED((npad, 16), jnp.float32),
            pltpu.VMEM_SHARED((npad, h), jnp.float32),
            pltpu.SemaphoreType.DMA,
            pltpu.SemaphoreType.DMA,
        ] + [pltpu.SemaphoreType.DMA] * (2 * NBUF),
    )
    def sca(xw_hbm, src_hbm, dst_hbm, agg_hbm, y1_hbm, deg_hbm,
            src_v, dst_v, dstp_v, rows_v, ones_v, zero16_v, deg_v, dinv_v,
            acc16_sh, acc64_sh, lsem, hsem, *sems):
        cid = lax.axis_index("c")
        sid = lax.axis_index("s")
        wid = sid * NC + cid
        pwid = sid * NC + (1 - cid)
        gsems = sems[:NBUF]
        ssems = sems[NBUF:]
        base = sid * rpt

        pltpu.async_copy(src_hbm.at[wid], src_v, lsem)
        pltpu.async_copy(dst_hbm.at[wid], dst_v, lsem)
        pltpu.async_copy(dst_hbm.at[pwid], dstp_v, lsem)

        def fill(r, carry):
            ones_v[r, :] = jnp.full((16,), 1.0, jnp.float32)
            zero16_v[r, :] = jnp.zeros((16,), jnp.float32)
            return carry
        lax.fori_loop(0, CHUNK, fill, 0)
        _zero_fill(rows_v.at[0], h)

        for z in range(zch):
            pltpu.sync_copy(zero16_v,
                            acc16_sh.at[pl.ds(base + z * CHUNK, CHUNK)])
            pltpu.sync_copy(rows_v.at[0],
                            acc64_sh.at[pl.ds(base + z * CHUNK, CHUNK)])
        pltpu.make_async_copy(src_hbm.at[wid], src_v, lsem).wait()
        pltpu.make_async_copy(dst_hbm.at[wid], dst_v, lsem).wait()
        pltpu.make_async_copy(dst_hbm.at[pwid], dstp_v, lsem).wait()
        plsc.subcore_barrier()

        for slab in (dst_v, dstp_v):
            def group(i, carry):
                for j in range(4):
                    pltpu.async_copy(ones_v, acc16_sh.at[slab.at[i * 4 + j]],
                                     hsem, add=True)
                for j in range(4):
                    pltpu.make_async_copy(
                        ones_v, acc16_sh.at[slab.at[i * 4 + j]], hsem).wait()
                return carry
            lax.fori_loop(0, epc // 4, group, 0)
        plsc.subcore_barrier()

        pltpu.sync_copy(acc16_sh.at[pl.ds(base, rpt)], deg_v)
        pltpu.sync_copy(deg_v, deg_hbm.at[cid, pl.ds(base, rpt)])
        _compute_dinv(deg_v, dinv_v, rpt)

        for z in range(zch):
            blk = rows_v.at[1]
            pltpu.sync_copy(xw_hbm.at[pl.ds(base + z * CHUNK, CHUNK)], blk)

            def scale(r, carry):
                sv = _splat(dinv_v, z * CHUNK + r)
                for q in range(h // 16):
                    blk[r, pl.ds(q * 16, 16)] = blk[r, pl.ds(q * 16, 16)] * sv
                return carry
            lax.fori_loop(0, CHUNK, scale, 0)
            pltpu.sync_copy(blk, y1_hbm.at[cid, pl.ds(base + z * CHUNK, CHUNK)])
        plsc.subcore_barrier()

        _emit_agg(y1_hbm.at[cid], src_v, dst_v, rows_v, acc64_sh,
                  gsems, ssems, epc)
        plsc.subcore_barrier()
        pltpu.sync_copy(acc64_sh.at[pl.ds(base, rpt)],
                        agg_hbm.at[cid, pl.ds(base, rpt)])

    return sca


def _agg_kernel(npad, epc, width):
    rpt = npad // NS
    zch = rpt // CHUNK

    @functools.partial(
        pl.kernel,
        out_type=jax.ShapeDtypeStruct((NC, npad, width), jnp.float32),
        mesh=_mesh(),
        compiler_params=_SC_PARAMS,
        scratch_types=[
            pltpu.VMEM((epc, CHUNK), jnp.int32),
            pltpu.VMEM((epc, CHUNK), jnp.int32),
            pltpu.VMEM((NBUF, CHUNK, width), jnp.float32),
            pltpu.VMEM_SHARED((npad, width), jnp.float32),
            pltpu.SemaphoreType.DMA,
        ] + [pltpu.SemaphoreType.DMA] * (2 * NBUF),
    )
    def agg(tab_hbm, src_hbm, dst_hbm, out_hbm,
            src_v, dst_v, rows_v, acc_sh, lsem, *sems):
        cid = lax.axis_index("c")
        sid = lax.axis_index("s")
        wid = sid * NC + cid
        gsems = sems[:NBUF]
        ssems = sems[NBUF:]
        base = sid * rpt

        pltpu.async_copy(src_hbm.at[wid], src_v, lsem)
        pltpu.async_copy(dst_hbm.at[wid], dst_v, lsem)

        _zero_fill(rows_v.at[0], width)
        for z in range(zch):
            pltpu.sync_copy(rows_v.at[0],
                            acc_sh.at[pl.ds(base + z * CHUNK, CHUNK)])
        pltpu.make_async_copy(src_hbm.at[wid], src_v, lsem).wait()
        pltpu.make_async_copy(dst_hbm.at[wid], dst_v, lsem).wait()
        plsc.subcore_barrier()

        _emit_agg(tab_hbm, src_v, dst_v, rows_v, acc_sh, gsems, ssems, epc)

        plsc.subcore_barrier()
        pltpu.sync_copy(acc_sh.at[pl.ds(base, rpt)],
                        out_hbm.at[cid, pl.ds(base, rpt)])

    return agg


def _tc1_body(x_ref, w_ref, o_ref):
    o_ref[...] = jnp.dot(x_ref[...], w_ref[...],
                         preferred_element_type=jnp.float32)


def _tcb_body(a0_ref, a1_ref, y1_ref, d_ref, b1_ref, o_ref):
    dinv = lax.rsqrt(d_ref[:, 0:1] + 1.0)
    u = a0_ref[...] + a1_ref[...] + y1_ref[...]
    h = jnp.maximum(dinv * u + b1_ref[...], 0.0)
    o_ref[...] = dinv * h


def _tcc_body(a0_ref, a1_ref, y2_ref, d_ref, wc_ref, bc_ref, o_ref):
    dinv = lax.rsqrt(d_ref[:, 0:1] + 1.0)
    z = dinv * (a0_ref[...] + a1_ref[...] + y2_ref[...])
    o_ref[...] = (jnp.dot(z, wc_ref[...], preferred_element_type=jnp.float32)
                  + bc_ref[...])


def kernel(x, edge_index, W1, b1, Wmu, bmu, Wls, bls):
    n, d_in = x.shape
    h_dim = W1.shape[1]
    out_dim = Wmu.shape[1]
    e = edge_index.shape[1]

    npad = _ceil_to(n + CHUNK, NS * CHUNK)
    epw = _ceil_to(-(-e // NW), 4 * CHUNK)
    epc = epw // CHUNK
    epad = epw * NW

    pad_idx = (n + (jnp.arange(epad - e, dtype=jnp.int32) % CHUNK))
    srcp = jnp.concatenate([edge_index[0], pad_idx]).reshape(NW, epc, CHUNK)
    dstp = jnp.concatenate([edge_index[1], pad_idx]).reshape(NW, epc, CHUNK)
    x_pad = jnp.zeros((npad, d_in), x.dtype).at[:n].set(x)

    br = 2048
    grid = (npad // br,)
    row_spec = lambda width: pl.BlockSpec((br, width), lambda i: (i, 0))
    full_spec = lambda shape: pl.BlockSpec(shape, lambda i: (0, 0))

    xw = pl.pallas_call(
        _tc1_body,
        grid=grid,
        in_specs=[row_spec(d_in), full_spec((d_in, h_dim))],
        out_specs=row_spec(h_dim),
        out_shape=jax.ShapeDtypeStruct((npad, h_dim), jnp.float32),
    )(x_pad, W1)

    a1p, y1c, degc = _sc_a_kernel(npad, epc, h_dim)(xw, srcp, dstp)

    y2 = pl.pallas_call(
        _tcb_body,
        grid=grid,
        in_specs=[row_spec(h_dim), row_spec(h_dim), row_spec(h_dim),
                  row_spec(16), full_spec((1, h_dim))],
        out_specs=row_spec(h_dim),
        out_shape=jax.ShapeDtypeStruct((npad, h_dim), jnp.float32),
    )(a1p[0], a1p[1], y1c[0], degc[0], b1.reshape(1, h_dim))

    a2p = _agg_kernel(npad, epc, h_dim)(y2, srcp, dstp)

    wc = jnp.concatenate([Wmu, Wls], axis=1)
    bc = jnp.concatenate([bmu, bls]).reshape(1, 2 * out_dim)
    out = pl.pallas_call(
        _tcc_body,
        grid=grid,
        in_specs=[row_spec(h_dim), row_spec(h_dim), row_spec(h_dim),
                  row_spec(16),
                  full_spec((h_dim, 2 * out_dim)), full_spec((1, 2 * out_dim))],
        out_specs=row_spec(2 * out_dim),
        out_shape=jax.ShapeDtypeStruct((npad, 2 * out_dim), jnp.float32),
    )(a2p[0], a2p[1], y2, degc[0], wc, bc)

    return out[:n, :out_dim], out[:n, out_dim:]


# speedup vs baseline: 1.2977x; 1.0871x over previous
import functools

import jax
import jax.numpy as jnp
from jax import lax
from jax.experimental import pallas as pl
from jax.experimental.pallas import tpu as pltpu
from jax.experimental.pallas import tpu_sc as plsc

NC = 2
NS = 16
NW = NC * NS
CHUNK = 128


def _ceil_to(a, m):
    return (a + m - 1) // m * m


def _mesh():
    return plsc.VectorSubcoreMesh(
        core_axis_name="c", subcore_axis_name="s",
        num_cores=NC, num_subcores=NS)


def _deg_kernel(npad, epc):
    rpt = npad // NS
    zch = rpt // CHUNK
    w = 16

    @functools.partial(
        pl.kernel,
        out_type=jax.ShapeDtypeStruct((NC, npad, w), jnp.float32),
        mesh=_mesh(),
        compiler_params=pltpu.CompilerParams(use_tc_tiling_on_sc=False),
        scratch_types=[
            pltpu.VMEM((epc, CHUNK), jnp.int32),
            pltpu.VMEM((CHUNK, w), jnp.float32),
            pltpu.VMEM((CHUNK, w), jnp.float32),
            pltpu.VMEM_SHARED((npad, w), jnp.float32),
            pltpu.SemaphoreType.DMA,
        ],
    )
    def deg(dst_hbm, out_hbm, idx_v, ones_v, zero_v, acc_sh, sem):
        cid = lax.axis_index("c")
        sid = lax.axis_index("s")
        wid = sid * NC + cid

        def fill(r, carry):
            ones_v[r, :] = jnp.full((16,), 1.0, jnp.float32)
            zero_v[r, :] = jnp.zeros((16,), jnp.float32)
            return carry
        lax.fori_loop(0, CHUNK, fill, 0)

        for z in range(zch):
            pltpu.sync_copy(zero_v,
                            acc_sh.at[pl.ds(sid * rpt + z * CHUNK, CHUNK)])
        pltpu.sync_copy(dst_hbm.at[wid], idx_v)
        plsc.subcore_barrier()

        def group(i, carry):
            for j in range(4):
                pltpu.async_copy(ones_v, acc_sh.at[idx_v.at[i * 4 + j]],
                                 sem, add=True)
            for j in range(4):
                pltpu.make_async_copy(ones_v, acc_sh.at[idx_v.at[i * 4 + j]],
                                      sem).wait()
            return carry
        lax.fori_loop(0, epc // 4, group, 0)

        plsc.subcore_barrier()
        pltpu.sync_copy(acc_sh.at[pl.ds(sid * rpt, rpt)],
                        out_hbm.at[cid, pl.ds(sid * rpt, rpt)])

    return deg


def _agg_kernel(npad, epc, width):
    rpt = npad // NS
    zch = rpt // CHUNK
    nbuf = 5
    rounds = epc // nbuf

    @functools.partial(
        pl.kernel,
        out_type=jax.ShapeDtypeStruct((NC, npad, width), jnp.float32),
        mesh=_mesh(),
        compiler_params=pltpu.CompilerParams(use_tc_tiling_on_sc=False),
        scratch_types=[
            pltpu.VMEM((epc, CHUNK), jnp.int32),
            pltpu.VMEM((epc, CHUNK), jnp.int32),
            pltpu.VMEM((nbuf, CHUNK, width), jnp.float32),
            pltpu.VMEM((CHUNK, width), jnp.float32),
            pltpu.VMEM_SHARED((npad, width), jnp.float32),
            pltpu.SemaphoreType.DMA,
        ] + [pltpu.SemaphoreType.DMA] * (2 * nbuf),
    )
    def agg(tab_hbm, src_hbm, dst_hbm, out_hbm,
            src_v, dst_v, rows_v, zero_v, acc_sh, lsem, *sems):
        cid = lax.axis_index("c")
        sid = lax.axis_index("s")
        wid = sid * NC + cid
        gsems = sems[:nbuf]
        ssems = sems[nbuf:]

        pltpu.async_copy(src_hbm.at[wid], src_v, lsem)
        pltpu.async_copy(dst_hbm.at[wid], dst_v, lsem)

        def fill(r, carry):
            for j in range(width // 16):
                zero_v[r, pl.ds(j * 16, 16)] = jnp.zeros((16,), jnp.float32)
            return carry
        lax.fori_loop(0, CHUNK, fill, 0)

        for z in range(zch):
            pltpu.sync_copy(zero_v,
                            acc_sh.at[pl.ds(sid * rpt + z * CHUNK, CHUNK)])
        pltpu.make_async_copy(src_hbm.at[wid], src_v, lsem).wait()
        pltpu.make_async_copy(dst_hbm.at[wid], dst_v, lsem).wait()
        plsc.subcore_barrier()

        def gstart(c, b):
            pltpu.async_copy(tab_hbm.at[src_v.at[c]], rows_v.at[b], gsems[b])

        def gwait(c, b):
            pltpu.make_async_copy(tab_hbm.at[src_v.at[c]], rows_v.at[b],
                                  gsems[b]).wait()

        def sstart(c, b):
            pltpu.async_copy(rows_v.at[b], acc_sh.at[dst_v.at[c]],
                             ssems[b], add=True)

        def swait(c, b):
            pltpu.make_async_copy(rows_v.at[b], acc_sh.at[dst_v.at[c]],
                                  ssems[b]).wait()

        for b in range(nbuf):
            gstart(b, b)

        def round_body(i, carry):
            for b in range(nbuf):
                c = nbuf * i + b
                gwait(c, b)
                sstart(c, b)
            for b in range(nbuf):
                c = nbuf * i + b

                @pl.when(c + nbuf < epc)
                def _():
                    swait(c, b)
                    gstart(c + nbuf, b)
            return carry
        lax.fori_loop(0, rounds, round_body, 0)
        for b in range(nbuf):
            swait(epc - nbuf + b, b)

        plsc.subcore_barrier()
        pltpu.sync_copy(acc_sh.at[pl.ds(sid * rpt, rpt)],
                        out_hbm.at[cid, pl.ds(sid * rpt, rpt)])

    return agg


def _dinv(d0_ref, d1_ref):
    return lax.rsqrt(d0_ref[:, 0:1] + d1_ref[:, 0:1] + 1.0)


def _tca_body(x_ref, w_ref, d0_ref, d1_ref, o_ref):
    xw = jnp.dot(x_ref[...], w_ref[...], preferred_element_type=jnp.float32)
    o_ref[...] = _dinv(d0_ref, d1_ref) * xw


def _tcb_body(a0_ref, a1_ref, y1_ref, d0_ref, d1_ref, b1_ref, o_ref):
    dinv = _dinv(d0_ref, d1_ref)
    u = a0_ref[...] + a1_ref[...] + y1_ref[...]
    h = jnp.maximum(dinv * u + b1_ref[...], 0.0)
    o_ref[...] = dinv * h


def _tcc_body(a0_ref, a1_ref, y2_ref, d0_ref, d1_ref, wc_ref, bc_ref, o_ref):
    dinv = _dinv(d0_ref, d1_ref)
    z = dinv * (a0_ref[...] + a1_ref[...] + y2_ref[...])
    o_ref[...] = (jnp.dot(z, wc_ref[...], preferred_element_type=jnp.float32)
                  + bc_ref[...])


def kernel(x, edge_index, W1, b1, Wmu, bmu, Wls, bls):
    n, d_in = x.shape
    h_dim = W1.shape[1]
    out_dim = Wmu.shape[1]
    e = edge_index.shape[1]

    npad = _ceil_to(n + CHUNK, NS * CHUNK)
    epw = _ceil_to(-(-e // NW), 20 * CHUNK)
    epc = epw // CHUNK
    epad = epw * NW

    pad_idx = (n + (jnp.arange(epad - e, dtype=jnp.int32) % CHUNK))
    srcp = jnp.concatenate([edge_index[0], pad_idx]).reshape(NW, epc, CHUNK)
    dstp = jnp.concatenate([edge_index[1], pad_idx]).reshape(NW, epc, CHUNK)
    x_pad = jnp.zeros((npad, d_in), x.dtype).at[:n].set(x)

    degp = _deg_kernel(npad, epc)(dstp)
    d0 = degp[0]
    d1 = degp[1]

    br = 2048
    grid = (npad // br,)
    row_spec = lambda width: pl.BlockSpec((br, width), lambda i: (i, 0))
    full_spec = lambda shape: pl.BlockSpec(shape, lambda i: (0, 0))

    y1 = pl.pallas_call(
        _tca_body,
        grid=grid,
        in_specs=[row_spec(d_in), full_spec((d_in, h_dim)),
                  row_spec(16), row_spec(16)],
        out_specs=row_spec(h_dim),
        out_shape=jax.ShapeDtypeStruct((npad, h_dim), jnp.float32),
    )(x_pad, W1, d0, d1)

    agg = _agg_kernel(npad, epc, h_dim)
    a1p = agg(y1, srcp, dstp)

    y2 = pl.pallas_call(
        _tcb_body,
        grid=grid,
        in_specs=[row_spec(h_dim), row_spec(h_dim), row_spec(h_dim),
                  row_spec(16), row_spec(16), full_spec((1, h_dim))],
        out_specs=row_spec(h_dim),
        out_shape=jax.ShapeDtypeStruct((npad, h_dim), jnp.float32),
    )(a1p[0], a1p[1], y1, d0, d1, b1.reshape(1, h_dim))

    a2p = agg(y2, srcp, dstp)

    wc = jnp.concatenate([Wmu, Wls], axis=1)
    bc = jnp.concatenate([bmu, bls]).reshape(1, 2 * out_dim)
    out = pl.pallas_call(
        _tcc_body,
        grid=grid,
        in_specs=[row_spec(h_dim), row_spec(h_dim), row_spec(h_dim),
                  row_spec(16), row_spec(16),
                  full_spec((h_dim, 2 * out_dim)), full_spec((1, 2 * out_dim))],
        out_specs=row_spec(2 * out_dim),
        out_shape=jax.ShapeDtypeStruct((npad, 2 * out_dim), jnp.float32),
    )(a2p[0], a2p[1], y2, d0, d1, wc, bc)

    return out[:n, :out_dim], out[:n, out_dim:]


# speedup vs baseline: 1.3028x; 1.0039x over previous
import functools

import jax
import jax.numpy as jnp
from jax import lax
from jax.experimental import pallas as pl
from jax.experimental.pallas import tpu as pltpu
from jax.experimental.pallas import tpu_sc as plsc

NC = 2
NS = 16
NW = NC * NS
CHUNK = 128


def _ceil_to(a, m):
    return (a + m - 1) // m * m


def _mesh():
    return plsc.VectorSubcoreMesh(
        core_axis_name="c", subcore_axis_name="s",
        num_cores=NC, num_subcores=NS)


def _deg_kernel(npad, epc):
    rpt = npad // NS
    zch = rpt // CHUNK
    w = 16

    @functools.partial(
        pl.kernel,
        out_type=jax.ShapeDtypeStruct((NC, npad, w), jnp.float32),
        mesh=_mesh(),
        compiler_params=pltpu.CompilerParams(use_tc_tiling_on_sc=False),
        scratch_types=[
            pltpu.VMEM((epc, CHUNK), jnp.int32),
            pltpu.VMEM((CHUNK, w), jnp.float32),
            pltpu.VMEM((CHUNK, w), jnp.float32),
            pltpu.VMEM_SHARED((npad, w), jnp.float32),
            pltpu.SemaphoreType.DMA,
        ],
    )
    def deg(dst_hbm, out_hbm, idx_v, ones_v, zero_v, acc_sh, sem):
        cid = lax.axis_index("c")
        sid = lax.axis_index("s")
        wid = sid * NC + cid

        def fill(r, carry):
            ones_v[r, :] = jnp.full((16,), 1.0, jnp.float32)
            zero_v[r, :] = jnp.zeros((16,), jnp.float32)
            return carry
        lax.fori_loop(0, CHUNK, fill, 0)

        for z in range(zch):
            pltpu.sync_copy(zero_v,
                            acc_sh.at[pl.ds(sid * rpt + z * CHUNK, CHUNK)])
        pltpu.sync_copy(dst_hbm.at[wid], idx_v)
        plsc.subcore_barrier()

        def group(i, carry):
            for j in range(4):
                pltpu.async_copy(ones_v, acc_sh.at[idx_v.at[i * 4 + j]],
                                 sem, add=True)
            for j in range(4):
                pltpu.make_async_copy(ones_v, acc_sh.at[idx_v.at[i * 4 + j]],
                                      sem).wait()
            return carry
        lax.fori_loop(0, epc // 4, group, 0)

        plsc.subcore_barrier()
        pltpu.sync_copy(acc_sh.at[pl.ds(sid * rpt, rpt)],
                        out_hbm.at[cid, pl.ds(sid * rpt, rpt)])

    return deg


def _agg_kernel(npad, epc, width):
    rpt = npad // NS
    zch = rpt // CHUNK
    nbuf = 5
    rounds = epc // nbuf

    @functools.partial(
        pl.kernel,
        out_type=jax.ShapeDtypeStruct((NC, npad, width), jnp.float32),
        mesh=_mesh(),
        compiler_params=pltpu.CompilerParams(use_tc_tiling_on_sc=False),
        scratch_types=[
            pltpu.VMEM((epc, CHUNK), jnp.int32),
            pltpu.VMEM((epc, CHUNK), jnp.int32),
            pltpu.VMEM((nbuf, CHUNK, width), jnp.float32),
            pltpu.VMEM((CHUNK, width), jnp.float32),
            pltpu.VMEM_SHARED((npad, width), jnp.float32),
            pltpu.SemaphoreType.DMA,
        ] + [pltpu.SemaphoreType.DMA] * (2 * nbuf),
    )
    def agg(tab_hbm, src_hbm, dst_hbm, out_hbm,
            src_v, dst_v, rows_v, zero_v, acc_sh, lsem, *sems):
        cid = lax.axis_index("c")
        sid = lax.axis_index("s")
        wid = sid * NC + cid
        gsems = sems[:nbuf]
        ssems = sems[nbuf:]

        pltpu.async_copy(src_hbm.at[wid], src_v, lsem)
        pltpu.async_copy(dst_hbm.at[wid], dst_v, lsem)

        def fill(r, carry):
            for j in range(width // 16):
                zero_v[r, pl.ds(j * 16, 16)] = jnp.zeros((16,), jnp.float32)
            return carry
        lax.fori_loop(0, CHUNK, fill, 0)

        for z in range(zch):
            pltpu.sync_copy(zero_v,
                            acc_sh.at[pl.ds(sid * rpt + z * CHUNK, CHUNK)])
        pltpu.make_async_copy(src_hbm.at[wid], src_v, lsem).wait()
        pltpu.make_async_copy(dst_hbm.at[wid], dst_v, lsem).wait()
        plsc.subcore_barrier()

        def gstart(c, b):
            pltpu.async_copy(tab_hbm.at[src_v.at[c]], rows_v.at[b], gsems[b])

        def gwait(c, b):
            pltpu.make_async_copy(tab_hbm.at[src_v.at[c]], rows_v.at[b],
                                  gsems[b]).wait()

        def sstart(c, b):
            pltpu.async_copy(rows_v.at[b], acc_sh.at[dst_v.at[c]],
                             ssems[b], add=True)

        def swait(c, b):
            pltpu.make_async_copy(rows_v.at[b], acc_sh.at[dst_v.at[c]],
                                  ssems[b]).wait()

        for b in range(nbuf):
            gstart(b, b)

        def round_body(i, carry):
            for b in range(nbuf):
                c = nbuf * i + b
                gwait(c, b)
                sstart(c, b)
            for b in range(nbuf):
                c = nbuf * i + b

                @pl.when(c + nbuf < epc)
                def _():
                    swait(c, b)
                    gstart(c + nbuf, b)
            return carry
        lax.fori_loop(0, rounds, round_body, 0)
        for b in range(nbuf):
            swait(epc - nbuf + b, b)

        plsc.subcore_barrier()
        pltpu.sync_copy(acc_sh.at[pl.ds(sid * rpt, rpt)],
                        out_hbm.at[cid, pl.ds(sid * rpt, rpt)])

    return agg


def _dinv(d0_ref, d1_ref):
    return lax.rsqrt(d0_ref[:, 0:1] + d1_ref[:, 0:1] + 1.0)


def _tc1_body(x_ref, w_ref, o_ref):
    o_ref[...] = jnp.dot(x_ref[...], w_ref[...],
                         preferred_element_type=jnp.float32)


def _tcs_body(xw_ref, d0_ref, d1_ref, o_ref):
    o_ref[...] = _dinv(d0_ref, d1_ref) * xw_ref[...]


def _tcb_body(a0_ref, a1_ref, y1_ref, d0_ref, d1_ref, b1_ref, o_ref):
    dinv = _dinv(d0_ref, d1_ref)
    u = a0_ref[...] + a1_ref[...] + y1_ref[...]
    h = jnp.maximum(dinv * u + b1_ref[...], 0.0)
    o_ref[...] = dinv * h


def _tcc_body(a0_ref, a1_ref, y2_ref, d0_ref, d1_ref, wc_ref, bc_ref, o_ref):
    dinv = _dinv(d0_ref, d1_ref)
    z = dinv * (a0_ref[...] + a1_ref[...] + y2_ref[...])
    o_ref[...] = (jnp.dot(z, wc_ref[...], preferred_element_type=jnp.float32)
                  + bc_ref[...])


def kernel(x, edge_index, W1, b1, Wmu, bmu, Wls, bls):
    n, d_in = x.shape
    h_dim = W1.shape[1]
    out_dim = Wmu.shape[1]
    e = edge_index.shape[1]

    npad = _ceil_to(n + CHUNK, NS * CHUNK)
    epw = _ceil_to(-(-e // NW), 20 * CHUNK)
    epc = epw // CHUNK
    epad = epw * NW

    pad_idx = (n + (jnp.arange(epad - e, dtype=jnp.int32) % CHUNK))
    srcp = jnp.concatenate([edge_index[0], pad_idx]).reshape(NW, epc, CHUNK)
    dstp = jnp.concatenate([edge_index[1], pad_idx]).reshape(NW, epc, CHUNK)
    x_pad = jnp.zeros((npad, d_in), x.dtype).at[:n].set(x)

    xw = pl.pallas_call(
        _tc1_body,
        grid=(npad // 2048,),
        in_specs=[pl.BlockSpec((2048, d_in), lambda i: (i, 0)),
                  pl.BlockSpec((d_in, h_dim), lambda i: (0, 0))],
        out_specs=pl.BlockSpec((2048, h_dim), lambda i: (i, 0)),
        out_shape=jax.ShapeDtypeStruct((npad, h_dim), jnp.float32),
    )(x_pad, W1)

    degp = _deg_kernel(npad, epc)(dstp)
    d0 = degp[0]
    d1 = degp[1]

    br = 2048
    grid = (npad // br,)
    row_spec = lambda width: pl.BlockSpec((br, width), lambda i: (i, 0))
    full_spec = lambda shape: pl.BlockSpec(shape, lambda i: (0, 0))

    y1 = pl.pallas_call(
        _tcs_body,
        grid=grid,
        in_specs=[row_spec(h_dim), row_spec(16), row_spec(16)],
        out_specs=row_spec(h_dim),
        out_shape=jax.ShapeDtypeStruct((npad, h_dim), jnp.float32),
    )(xw, d0, d1)

    agg = _agg_kernel(npad, epc, h_dim)
    a1p = agg(y1, srcp, dstp)

    y2 = pl.pallas_call(
        _tcb_body,
        grid=grid,
        in_specs=[row_spec(h_dim), row_spec(h_dim), row_spec(h_dim),
                  row_spec(16), row_spec(16), full_spec((1, h_dim))],
        out_specs=row_spec(h_dim),
        out_shape=jax.ShapeDtypeStruct((npad, h_dim), jnp.float32),
    )(a1p[0], a1p[1], y1, d0, d1, b1.reshape(1, h_dim))

    a2p = agg(y2, srcp, dstp)

    wc = jnp.concatenate([Wmu, Wls], axis=1)
    bc = jnp.concatenate([bmu, bls]).reshape(1, 2 * out_dim)
    out = pl.pallas_call(
        _tcc_body,
        grid=grid,
        in_specs=[row_spec(h_dim), row_spec(h_dim), row_spec(h_dim),
                  row_spec(16), row_spec(16),
                  full_spec((h_dim, 2 * out_dim)), full_spec((1, 2 * out_dim))],
        out_specs=row_spec(2 * out_dim),
        out_shape=jax.ShapeDtypeStruct((npad, 2 * out_dim), jnp.float32),
    )(a2p[0], a2p[1], y2, d0, d1, wc, bc)

    return out[:n, :out_dim], out[:n, out_dim:]
